# baseline (speedup 1.0000x reference)
import jax
import jax.numpy as jnp
from jax import lax
from jax.experimental import pallas as pl


def _ctx_kernel(ht_ref, x1_ref, w1b_ref, wc_ref, bc_ref, b1_ref,
                ctx_ref, u1_ref, cu_ref):
    d = w1b_ref.shape[0]
    num_case = ht_ref.shape[0]
    S = jnp.dot(ht_ref[...], x1_ref[...], preferred_element_type=jnp.float32)
    ctx = S[:, :d] / jnp.maximum(S[:, d:d + 1], 1e-8)
    u1 = jnp.dot(ctx, w1b_ref[...],
                 preferred_element_type=jnp.float32) + b1_ref[...]
    cu = jnp.dot(ctx, wc_ref[...],
                 preferred_element_type=jnp.float32) + bc_ref[...]
    rep = (num_case, 8, d)
    ctx_ref[...] = jnp.broadcast_to(ctx[:, None, :], rep).astype(jnp.bfloat16)
    u1_ref[...] = jnp.broadcast_to(u1[:, None, :], rep)
    cu_ref[...] = jnp.broadcast_to(cu[:, None, :], rep)


def _edge_kernel(x_ref, ctx_ref, u1_ref, cu_ref, w1a_ref, w1c_ref, w1d_ref,
                 w2_ref, b2_ref, lng_ref, lnb_ref, out_ref):
    X = x_ref[...]
    nb, d = X.shape
    X3 = X.reshape(nb // 8, 8, d)
    Xb = X.astype(jnp.bfloat16)
    X3b = Xb.reshape(nb // 8, 8, d)
    P = jnp.dot(Xb, w1a_ref[...], preferred_element_type=jnp.float32)
    num_case = ctx_ref.shape[0]
    ones_d = jnp.full((d, d), 1.0 / d, dtype=jnp.float32)

    for j in range(num_case):
        cb = ctx_ref[j, :, :][None]
        u1 = u1_ref[j, :, :][None]
        cu = cu_ref[j, :, :][None]
        Y1 = (X3b * cb).reshape(nb, d)
        Y2 = jnp.abs(X3b - cb).reshape(nb, d)
        G = (P
             + jnp.dot(Y1, w1c_ref[...], preferred_element_type=jnp.float32)
             + jnp.dot(Y2, w1d_ref[...], preferred_element_type=jnp.float32)
             ).reshape(nb // 8, 8, d) + u1
        h = jnp.maximum(G.reshape(nb, d), 0.0).astype(jnp.bfloat16)
        logits = jnp.dot(h, w2_ref[...],
                         preferred_element_type=jnp.float32) + b2_ref[...]
        t = 0.3 * jax.nn.sigmoid(logits)
        pre = X + (t.reshape(nb // 8, 8, d) * (cu - X3)).reshape(nb, d)
        mu = jnp.dot(pre, ones_d, preferred_element_type=jnp.float32)
        msq = jnp.dot(pre * pre, ones_d, preferred_element_type=jnp.float32)
        var = msq - mu * mu
        out = (pre - mu) * lax.rsqrt(var + 1e-5) * lng_ref[...] + lnb_ref[...]
        out_ref[:, j, :] = out


def kernel(node_repr, H_case, Wc, bc, W1, b1, W2, b2, ln_g, ln_b):
    num_hpo, d = node_repr.shape
    num_case = H_case.shape[1]
    dm = W1.shape[1]

    X1 = jnp.concatenate(
        [node_repr, jnp.ones((num_hpo, 1), dtype=node_repr.dtype)], axis=1)
    Ht = H_case.T

    ctx, u1, cu = pl.pallas_call(
        _ctx_kernel,
        out_shape=[
            jax.ShapeDtypeStruct((num_case, 8, d), jnp.bfloat16),
            jax.ShapeDtypeStruct((num_case, 8, dm), jnp.float32),
            jax.ShapeDtypeStruct((num_case, 8, d), jnp.float32),
        ],
    )(Ht, X1, W1[d:2 * d], Wc, bc.reshape(1, d), b1.reshape(1, dm))

    bf = jnp.bfloat16
    NB = 400
    grid = (num_hpo // NB,)
    out = pl.pallas_call(
        _edge_kernel,
        grid=grid,
        in_specs=[
            pl.BlockSpec((NB, d), lambda b: (b, 0)),
            pl.BlockSpec((num_case, 8, d), lambda b: (0, 0, 0)),
            pl.BlockSpec((num_case, 8, dm), lambda b: (0, 0, 0)),
            pl.BlockSpec((num_case, 8, d), lambda b: (0, 0, 0)),
            pl.BlockSpec((d, dm), lambda b: (0, 0)),
            pl.BlockSpec((d, dm), lambda b: (0, 0)),
            pl.BlockSpec((d, dm), lambda b: (0, 0)),
            pl.BlockSpec((dm, d), lambda b: (0, 0)),
            pl.BlockSpec((1, d), lambda b: (0, 0)),
            pl.BlockSpec((1, d), lambda b: (0, 0)),
            pl.BlockSpec((1, d), lambda b: (0, 0)),
        ],
        out_specs=pl.BlockSpec((NB, num_case, d), lambda b: (b, 0, 0)),
        out_shape=jax.ShapeDtypeStruct((num_hpo, num_case, d), jnp.float32),
    )(node_repr, ctx, u1, cu,
      W1[:d].astype(bf), W1[2 * d:3 * d].astype(bf), W1[3 * d:].astype(bf),
      W2.astype(bf),
      b2.reshape(1, d), ln_g.reshape(1, d), ln_b.reshape(1, d))

    return out.reshape(num_hpo * num_case, d)


# speedup vs baseline: 1.5180x; 1.5180x over previous
import jax
import jax.numpy as jnp
from jax import lax
from jax.experimental import pallas as pl

_GROUP = 8


def _ctx_kernel(ht_ref, x1_ref, w1b_ref, wc_ref, bc_ref, b1_ref,
                ctxt_ref, u1t_ref, cut_ref):
    d = w1b_ref.shape[0]
    num_case = ht_ref.shape[0]
    S = jnp.dot(ht_ref[...], x1_ref[...], preferred_element_type=jnp.float32)
    ctx = S[:, :d] / jnp.maximum(S[:, d:d + 1], 1e-8)
    u1 = jnp.dot(ctx, w1b_ref[...],
                 preferred_element_type=jnp.float32) + b1_ref[...]
    cu = jnp.dot(ctx, wc_ref[...],
                 preferred_element_type=jnp.float32) + bc_ref[...]
    rep = (_GROUP, num_case, d)
    ctxt_ref[...] = jnp.broadcast_to(ctx[None], rep).reshape(
        _GROUP * num_case, d)
    u1t_ref[...] = jnp.broadcast_to(u1[None], rep).reshape(
        _GROUP * num_case, d)
    cut_ref[...] = jnp.broadcast_to(cu[None], rep).reshape(
        _GROUP * num_case, d)


def _edge_kernel(x_ref, ctxt_ref, u1t_ref, cut_ref, w1a_ref, w1c_ref,
                 w1d_ref, w2_ref, b2_ref, lng_ref, lnb_ref, out_ref):
    nb, d = x_ref.shape
    rows = ctxt_ref.shape[0]
    X = x_ref[...]
    P = jnp.dot(X, w1a_ref[...], preferred_element_type=jnp.float32)
    Ct = ctxt_ref[...]
    U1t = u1t_ref[...]
    Cut = cut_ref[...]
    num_case = rows // _GROUP
    B = (lax.broadcasted_iota(jnp.int32, (rows, _GROUP), 0) // num_case ==
         lax.broadcasted_iota(jnp.int32, (rows, _GROUP), 1)
         ).astype(jnp.float32)
    ones_d = jnp.full((d, d), 1.0 / d, dtype=jnp.float32)

    for g in range(nb // _GROUP):
        Xg = X[g * _GROUP:(g + 1) * _GROUP, :]
        Pg = P[g * _GROUP:(g + 1) * _GROUP, :]
        Ex = jnp.dot(B, Xg, preferred_element_type=jnp.float32)
        Ep = jnp.dot(B, Pg, preferred_element_type=jnp.float32)
        Y1 = Ex * Ct
        Y2 = jnp.abs(Ex - Ct)
        G = (Ep + U1t
             + jnp.dot(Y1, w1c_ref[...], preferred_element_type=jnp.float32)
             + jnp.dot(Y2, w1d_ref[...], preferred_element_type=jnp.float32))
        h = jnp.maximum(G, 0.0)
        logits = jnp.dot(h, w2_ref[...],
                         preferred_element_type=jnp.float32) + b2_ref[...]
        t = 0.3 * jax.nn.sigmoid(logits)
        pre = Ex + t * (Cut - Ex)
        mu = jnp.dot(pre, ones_d, preferred_element_type=jnp.float32)
        msq = jnp.dot(pre * pre, ones_d, preferred_element_type=jnp.float32)
        var = msq - mu * mu
        out = (pre - mu) * lax.rsqrt(var + 1e-5) * lng_ref[...] + lnb_ref[...]
        out_ref[pl.ds(g * rows, rows), :] = out


def kernel(node_repr, H_case, Wc, bc, W1, b1, W2, b2, ln_g, ln_b):
    num_hpo, d = node_repr.shape
    num_case = H_case.shape[1]
    dm = W1.shape[1]
    rows = _GROUP * num_case

    X1 = jnp.concatenate(
        [node_repr, jnp.ones((num_hpo, 1), dtype=node_repr.dtype)], axis=1)
    Ht = H_case.T

    ctxt, u1t, cut = pl.pallas_call(
        _ctx_kernel,
        out_shape=[
            jax.ShapeDtypeStruct((rows, d), jnp.float32),
            jax.ShapeDtypeStruct((rows, dm), jnp.float32),
            jax.ShapeDtypeStruct((rows, d), jnp.float32),
        ],
    )(Ht, X1, W1[d:2 * d], Wc, bc.reshape(1, d), b1.reshape(1, dm))

    NB = 400
    grid = (num_hpo // NB,)
    out = pl.pallas_call(
        _edge_kernel,
        grid=grid,
        in_specs=[
            pl.BlockSpec((NB, d), lambda b: (b, 0)),
            pl.BlockSpec((rows, d), lambda b: (0, 0)),
            pl.BlockSpec((rows, dm), lambda b: (0, 0)),
            pl.BlockSpec((rows, d), lambda b: (0, 0)),
            pl.BlockSpec((d, dm), lambda b: (0, 0)),
            pl.BlockSpec((d, dm), lambda b: (0, 0)),
            pl.BlockSpec((d, dm), lambda b: (0, 0)),
            pl.BlockSpec((dm, d), lambda b: (0, 0)),
            pl.BlockSpec((1, d), lambda b: (0, 0)),
            pl.BlockSpec((1, d), lambda b: (0, 0)),
            pl.BlockSpec((1, d), lambda b: (0, 0)),
        ],
        out_specs=pl.BlockSpec((NB * num_case, d), lambda b: (b, 0)),
        out_shape=jax.ShapeDtypeStruct((num_hpo * num_case, d), jnp.float32),
    )(node_repr, ctxt, u1t, cut, W1[:d], W1[2 * d:3 * d], W1[3 * d:], W2,
      b2.reshape(1, d), ln_g.reshape(1, d), ln_b.reshape(1, d))

    return out


# speedup vs baseline: 2.5011x; 1.6476x over previous
import jax
import jax.numpy as jnp
from jax import lax
from jax.experimental import pallas as pl

_GROUP = 80


def _ctx_kernel(h_ref, x_ref, w1_ref, wc_ref, bc_ref, b1_ref,
                ctxt_ref, u1_ref, cut_ref):
    d = x_ref.shape[1]
    num_case = h_ref.shape[1]
    H = h_ref[...]
    X = x_ref[...]
    S = lax.dot_general(H, X, (((0,), (0,)), ((), ())),
                        preferred_element_type=jnp.float32)
    ws = lax.dot_general(H, jnp.ones((H.shape[0], 1), jnp.float32),
                         (((0,), (0,)), ((), ())),
                         preferred_element_type=jnp.float32)
    ctx = S / jnp.maximum(ws, 1e-8)
    u1 = jnp.dot(ctx, w1_ref[d:2 * d, :],
                 preferred_element_type=jnp.float32) + b1_ref[...]
    cu = jnp.dot(ctx, wc_ref[...],
                 preferred_element_type=jnp.float32) + bc_ref[...]
    rep = (_GROUP, num_case, d)
    ctxt_ref[...] = jnp.broadcast_to(ctx[None], rep).reshape(
        _GROUP * num_case, d)
    u1_ref[...] = u1
    cut_ref[...] = jnp.broadcast_to(cu[None], rep).reshape(
        _GROUP * num_case, d)


def _edge_kernel(x_ref, ctxt_ref, u1_ref, cut_ref, w1_ref,
                 w2_ref, b2_ref, lng_ref, lnb_ref, out_ref):
    nb, d = x_ref.shape
    rows = ctxt_ref.shape[0]
    num_case = rows // _GROUP
    X = x_ref[...]
    P = jnp.dot(X, w1_ref[0:d, :], preferred_element_type=jnp.float32)
    W1c = w1_ref[2 * d:3 * d, :]
    W1d = w1_ref[3 * d:4 * d, :]
    Ct = ctxt_ref[...]
    Cut = cut_ref[...]
    kdim = -(-(_GROUP + num_case) // 8) * 8
    r0 = lax.broadcasted_iota(jnp.int32, (rows, kdim), 0)
    r1 = lax.broadcasted_iota(jnp.int32, (rows, kdim), 1)
    A = ((r0 // num_case == r1).astype(jnp.float32)
         + ((r0 % num_case == r1 - _GROUP) & (r1 >= _GROUP)
            & (r1 < _GROUP + num_case)).astype(jnp.float32))
    Zl = jnp.concatenate(
        [jnp.concatenate([jnp.zeros((num_case, d), jnp.float32),
                          u1_ref[...]], axis=1),
         jnp.zeros((kdim - _GROUP - num_case, 2 * d), jnp.float32)], axis=0)
    ones_d = jnp.full((d, d), 1.0 / d, dtype=jnp.float32)

    for g in range(nb // _GROUP):
        Xg = X[g * _GROUP:(g + 1) * _GROUP, :]
        Pg = P[g * _GROUP:(g + 1) * _GROUP, :]
        Z = jnp.concatenate(
            [jnp.concatenate([Xg, Pg], axis=1), Zl], axis=0)
        R = jnp.dot(A, Z, preferred_element_type=jnp.float32)
        Ex = R[:, :d]
        EpU = R[:, d:]
        Y1 = Ex * Ct
        Y2 = jnp.abs(Ex - Ct)
        G = (EpU
             + jnp.dot(Y1, W1c, preferred_element_type=jnp.float32)
             + jnp.dot(Y2, W1d, preferred_element_type=jnp.float32))
        h = jnp.maximum(G, 0.0)
        logits = jnp.dot(h, w2_ref[...],
                         preferred_element_type=jnp.float32) + b2_ref[...]
        t = 0.3 * jax.nn.sigmoid(logits)
        pre = Ex + t * (Cut - Ex)
        mu = jnp.dot(pre, ones_d, preferred_element_type=jnp.float32)
        msq = jnp.dot(pre * pre, ones_d, preferred_element_type=jnp.float32)
        var = msq - mu * mu
        out = (pre - mu) * lax.rsqrt(var + 1e-5) * lng_ref[...] + lnb_ref[...]
        out_ref[pl.ds(g * rows, rows), :] = out


def kernel(node_repr, H_case, Wc, bc, W1, b1, W2, b2, ln_g, ln_b):
    num_hpo, d = node_repr.shape
    num_case = H_case.shape[1]
    dm = W1.shape[1]
    rows = _GROUP * num_case

    ctxt, u1, cut = pl.pallas_call(
        _ctx_kernel,
        out_shape=[
            jax.ShapeDtypeStruct((rows, d), jnp.float32),
            jax.ShapeDtypeStruct((num_case, dm), jnp.float32),
            jax.ShapeDtypeStruct((rows, d), jnp.float32),
        ],
    )(H_case, node_repr, W1, Wc, bc.reshape(1, d), b1.reshape(1, dm))

    NB = 400
    grid = (num_hpo // NB,)
    out = pl.pallas_call(
        _edge_kernel,
        grid=grid,
        in_specs=[
            pl.BlockSpec((NB, d), lambda b: (b, 0)),
            pl.BlockSpec((rows, d), lambda b: (0, 0)),
            pl.BlockSpec((num_case, dm), lambda b: (0, 0)),
            pl.BlockSpec((rows, d), lambda b: (0, 0)),
            pl.BlockSpec((4 * d, dm), lambda b: (0, 0)),
            pl.BlockSpec((dm, d), lambda b: (0, 0)),
            pl.BlockSpec((1, d), lambda b: (0, 0)),
            pl.BlockSpec((1, d), lambda b: (0, 0)),
            pl.BlockSpec((1, d), lambda b: (0, 0)),
        ],
        out_specs=pl.BlockSpec((NB * num_case, d), lambda b: (b, 0)),
        out_shape=jax.ShapeDtypeStruct((num_hpo * num_case, d), jnp.float32),
    )(node_repr, ctxt, u1, cut, W1, W2,
      b2.reshape(1, d), ln_g.reshape(1, d), ln_b.reshape(1, d))

    return out
